# baseline (speedup 1.0000x reference)
import jax
import jax.numpy as jnp
from jax import lax
from jax.experimental import pallas as pl
from jax.experimental.pallas import tpu as pltpu
from jax.experimental.pallas import tpu_sc as plsc

N = 30
E = N * (N - 1)
L = 16
NW = 32
EPT = 28
ETAB = NW * EPT
LAST = 31
LSTART = E - EPT


def _edge_table_body(rel_rec_hbm, rel_send_hbm, adj_hbm, tab_hbm,
                     recbuf, sendbuf, adjbuf, pairbuf):
    c = lax.axis_index("c")
    s = lax.axis_index("s")
    wid = s * 2 + c
    is_last = wid == LAST
    start = jnp.where(is_last, LSTART, wid * EPT)

    pltpu.sync_copy(rel_rec_hbm.at[pl.ds(start, EPT)], recbuf)
    pltpu.sync_copy(rel_send_hbm.at[pl.ds(start, EPT)], sendbuf)
    pltpu.sync_copy(adj_hbm, adjbuf)

    lane = lax.iota(jnp.int32, L)

    for base in (0, EPT - L):
        rows = lane + base
        rec_f = jnp.zeros((L,), jnp.float32)
        send_f = jnp.zeros((L,), jnp.float32)
        for n in range(N):
            col = jnp.full((L,), n, jnp.int32)
            rec_f = rec_f + plsc.load_gather(recbuf, [rows, col]) * float(n)
            send_f = send_f + plsc.load_gather(sendbuf, [rows, col]) * float(n)
        rec_i = rec_f.astype(jnp.int32)
        send_i = send_f.astype(jnp.int32)
        vals = plsc.load_gather(adjbuf, [send_i, rec_i])
        t = jnp.where(vals != 0.0,
                      jnp.full((L,), 1.0, jnp.float32),
                      jnp.full((L,), 0.0, jnp.float32))
        two_rows = rows + rows
        plsc.store_scatter(pairbuf, [two_rows], 1.0 - t)
        plsc.store_scatter(pairbuf, [two_rows + 1], t)
        if base == EPT - L:
            @pl.when(is_last)
            def _():
                mask = rows >= EPT - 2
                pos = two_rows - 2 * (EPT - 2)
                plsc.store_scatter(pairbuf, [pos], 1.0 - t, mask=mask)
                plsc.store_scatter(pairbuf, [pos + 1], t, mask=mask)

    @pl.when(jnp.logical_not(is_last))
    def _():
        pltpu.sync_copy(pairbuf.at[pl.ds(0, 2 * EPT)],
                        tab_hbm.at[pl.ds(2 * EPT * wid, 2 * EPT)])

    @pl.when(is_last)
    def _():
        pltpu.sync_copy(pairbuf.at[pl.ds(0, 8)],
                        tab_hbm.at[pl.ds(2 * (E - 2), 8)])


def _bcast_body(tab_ref, out_ref):
    row = tab_ref[pl.ds(0, 2 * E)]
    out_ref[...] = jnp.broadcast_to(row[None, :], out_ref.shape)


def kernel(inputs, weather, rel_rec, rel_send, adj_matrix):
    b = inputs.shape[0]
    mesh = plsc.VectorSubcoreMesh(core_axis_name="c", subcore_axis_name="s")
    sc = pl.kernel(
        _edge_table_body,
        out_type=jax.ShapeDtypeStruct((2 * ETAB,), jnp.float32),
        mesh=mesh,
        scratch_types=[
            pltpu.VMEM((EPT, N), jnp.float32),
            pltpu.VMEM((EPT, N), jnp.float32),
            pltpu.VMEM((N, N), jnp.float32),
            pltpu.VMEM((2 * EPT,), jnp.float32),
        ],
        compiler_params=pltpu.CompilerParams(
            use_tc_tiling_on_sc=False, needs_layout_passes=False),
    )
    tab = sc(rel_rec, rel_send, adj_matrix)

    bb = 32
    out = pl.pallas_call(
        _bcast_body,
        out_shape=jax.ShapeDtypeStruct((b, 2 * E), jnp.float32),
        grid=(b // bb,),
        in_specs=[pl.BlockSpec((2 * ETAB,), lambda i: (0,))],
        out_specs=pl.BlockSpec((bb, 2 * E), lambda i: (i, 0)),
    )(tab)
    return out.reshape(b, E, 2)


# speedup vs baseline: 1.0139x; 1.0139x over previous
import jax
import jax.numpy as jnp
from jax import lax
from jax.experimental import pallas as pl
from jax.experimental.pallas import tpu as pltpu
from jax.experimental.pallas import tpu_sc as plsc

N = 30
E = N * (N - 1)
L = 16
NW = 32
EPT = 28
ETAB = NW * EPT
LAST = 31
LSTART = E - EPT


def _edge_table_body(rel_rec_hbm, rel_send_hbm, adj_hbm, tab_hbm,
                     recbuf, sendbuf, adjbuf, pairbuf):
    c = lax.axis_index("c")
    s = lax.axis_index("s")
    wid = s * 2 + c
    is_last = wid == LAST
    start = jnp.where(is_last, LSTART, wid * EPT)

    pltpu.sync_copy(rel_rec_hbm.at[pl.ds(start, EPT)], recbuf)
    pltpu.sync_copy(rel_send_hbm.at[pl.ds(start, EPT)], sendbuf)
    pltpu.sync_copy(adj_hbm, adjbuf)

    lane = lax.iota(jnp.int32, L)

    for base in (0, EPT - L):
        rows = lane + base
        rec_f = jnp.zeros((L,), jnp.float32)
        send_f = jnp.zeros((L,), jnp.float32)
        for n in range(N):
            col = jnp.full((L,), n, jnp.int32)
            rec_f = rec_f + plsc.load_gather(recbuf, [rows, col]) * float(n)
            send_f = send_f + plsc.load_gather(sendbuf, [rows, col]) * float(n)
        rec_i = rec_f.astype(jnp.int32)
        send_i = send_f.astype(jnp.int32)
        vals = plsc.load_gather(adjbuf, [send_i, rec_i])
        t = jnp.where(vals != 0.0,
                      jnp.full((L,), 1.0, jnp.float32),
                      jnp.full((L,), 0.0, jnp.float32))
        two_rows = rows + rows
        plsc.store_scatter(pairbuf, [two_rows], 1.0 - t)
        plsc.store_scatter(pairbuf, [two_rows + 1], t)
        if base == EPT - L:
            @pl.when(is_last)
            def _():
                mask = rows >= EPT - 2
                pos = two_rows - 2 * (EPT - 2)
                plsc.store_scatter(pairbuf, [pos], 1.0 - t, mask=mask)
                plsc.store_scatter(pairbuf, [pos + 1], t, mask=mask)

    @pl.when(jnp.logical_not(is_last))
    def _():
        pltpu.sync_copy(pairbuf.at[pl.ds(0, 2 * EPT)],
                        tab_hbm.at[pl.ds(2 * EPT * wid, 2 * EPT)])

    @pl.when(is_last)
    def _():
        pltpu.sync_copy(pairbuf.at[pl.ds(0, 8)],
                        tab_hbm.at[pl.ds(2 * (E - 2), 8)])


def _bcast_body(tab_ref, out_ref):
    row = tab_ref[pl.ds(0, 2 * E)]
    out_ref[...] = jnp.broadcast_to(row[None, :], out_ref.shape)


def kernel(inputs, weather, rel_rec, rel_send, adj_matrix):
    b = inputs.shape[0]
    mesh = plsc.VectorSubcoreMesh(core_axis_name="c", subcore_axis_name="s")
    sc = pl.kernel(
        _edge_table_body,
        out_type=jax.ShapeDtypeStruct((2 * ETAB,), jnp.float32),
        mesh=mesh,
        scratch_types=[
            pltpu.VMEM((EPT, N), jnp.float32),
            pltpu.VMEM((EPT, N), jnp.float32),
            pltpu.VMEM((N, N), jnp.float32),
            pltpu.VMEM((2 * EPT,), jnp.float32),
        ],
        compiler_params=pltpu.CompilerParams(
            use_tc_tiling_on_sc=False, needs_layout_passes=False),
    )
    tab = sc(rel_rec, rel_send, adj_matrix)

    bb = 128
    out = pl.pallas_call(
        _bcast_body,
        out_shape=jax.ShapeDtypeStruct((b, 2 * E), jnp.float32),
        grid=(b // bb,),
        in_specs=[pl.BlockSpec((2 * ETAB,), lambda i: (0,))],
        out_specs=pl.BlockSpec((bb, 2 * E), lambda i: (i, 0)),
    )(tab)
    return out.reshape(b, E, 2)
